# baseline (speedup 1.0000x reference)
import jax
import jax.numpy as jnp
from jax.experimental import pallas as pl

NUM_GROUP = 512
GROUP_SIZE = 32


def _fps_one(pts, K):
    N = pts.shape[0]

    def body(i, carry):
        sel, dists, last = carry
        d = jnp.sum((pts - pts[last]) ** 2, axis=-1)
        dists = jnp.minimum(dists, d)
        nxt = jnp.argmax(dists).astype(jnp.int32)
        sel = sel.at[i].set(nxt)
        return (sel, dists, nxt)

    sel0 = jnp.zeros((K,), dtype=jnp.int32)
    dists0 = jnp.full((N,), jnp.inf, dtype=pts.dtype)
    sel, _, _ = jax.lax.fori_loop(1, K, body, (sel0, dists0, jnp.int32(0)))
    return sel


def _center_sub_kernel(nbr_ref, ctr_ref, out_ref):
    out_ref[...] = nbr_ref[...] - ctr_ref[...]


def kernel(xyz):
    B, N, _ = xyz.shape
    fps_idx = jax.vmap(lambda p: _fps_one(p, NUM_GROUP))(xyz)
    center = jax.vmap(lambda p, i: p[i])(xyz, fps_idx)
    d2 = (
        jnp.sum(center ** 2, axis=-1)[:, :, None]
        + jnp.sum(xyz ** 2, axis=-1)[:, None, :]
        - 2.0 * jnp.einsum("bgd,bnd->bgn", center, xyz)
    )
    _, idx = jax.lax.top_k(-d2, GROUP_SIZE)
    neighborhood_org = jax.vmap(lambda p, i: p[i])(xyz, idx)

    nbr_flat = neighborhood_org.reshape(B, NUM_GROUP, GROUP_SIZE * 3)
    ctr_b = jnp.tile(center, (1, 1, GROUP_SIZE)).reshape(
        B, NUM_GROUP, GROUP_SIZE * 3)
    neighborhood = pl.pallas_call(
        _center_sub_kernel,
        out_shape=jax.ShapeDtypeStruct((B, NUM_GROUP, GROUP_SIZE * 3),
                                       jnp.float32),
        grid=(B,),
        in_specs=[
            pl.BlockSpec((1, NUM_GROUP, GROUP_SIZE * 3), lambda b: (b, 0, 0)),
            pl.BlockSpec((1, NUM_GROUP, GROUP_SIZE * 3), lambda b: (b, 0, 0)),
        ],
        out_specs=pl.BlockSpec((1, NUM_GROUP, GROUP_SIZE * 3),
                               lambda b: (b, 0, 0)),
    )(nbr_flat, ctr_b).reshape(B, NUM_GROUP, GROUP_SIZE, 3)
    return (neighborhood, center, neighborhood_org)


# speedup vs baseline: 1.3020x; 1.3020x over previous
import jax
import jax.numpy as jnp
from jax.experimental import pallas as pl
from jax.experimental.pallas import tpu as pltpu

NUM_GROUP = 512
GROUP_SIZE = 32
NROW = 64
NCOL = 128


def _fps_kernel(xr_ref, center_ref):
    xa = xr_ref[0, 0]
    ya = xr_ref[0, 1]
    za = xr_ref[0, 2]
    flat = (jax.lax.broadcasted_iota(jnp.int32, (NROW, NCOL), 0) * NCOL
            + jax.lax.broadcasted_iota(jnp.int32, (NROW, NCOL), 1))
    lane = jax.lax.broadcasted_iota(jnp.int32, (1, NCOL), 1)

    def extract(ref, coord, r, l):
        row = ref[0, coord, pl.ds(r, 1), :]
        return jnp.sum(jnp.where(lane == l, row, 0.0))

    cx0 = extract(xr_ref, 0, 0, 0)
    cy0 = extract(xr_ref, 1, 0, 0)
    cz0 = extract(xr_ref, 2, 0, 0)
    center_ref[0, pl.ds(0, 1), pl.ds(0, 1)] = cx0[None, None]
    center_ref[0, pl.ds(0, 1), pl.ds(1, 1)] = cy0[None, None]
    center_ref[0, pl.ds(0, 1), pl.ds(2, 1)] = cz0[None, None]

    def body(i, carry):
        dmin, cx, cy, cz = carry
        dx = xa - cx
        dy = ya - cy
        dz = za - cz
        d = dx * dx + dy * dy + dz * dz
        dmin = jnp.minimum(dmin, d)
        m = jnp.max(dmin)
        cand = jnp.where(dmin == m, flat, jnp.int32(1 << 30))
        nxt = jnp.min(cand)
        r = nxt // NCOL
        l = nxt - r * NCOL
        ncx = extract(xr_ref, 0, r, l)
        ncy = extract(xr_ref, 1, r, l)
        ncz = extract(xr_ref, 2, r, l)
        center_ref[0, pl.ds(i, 1), pl.ds(0, 1)] = ncx[None, None]
        center_ref[0, pl.ds(i, 1), pl.ds(1, 1)] = ncy[None, None]
        center_ref[0, pl.ds(i, 1), pl.ds(2, 1)] = ncz[None, None]
        return (dmin, ncx, ncy, ncz)

    dmin0 = jnp.full((NROW, NCOL), jnp.inf, dtype=jnp.float32)
    jax.lax.fori_loop(1, NUM_GROUP, body, (dmin0, cx0, cy0, cz0))


def _fps_centers(xyz):
    B = xyz.shape[0]
    xr = xyz.transpose(0, 2, 1).reshape(B, 3, NROW, NCOL)
    return pl.pallas_call(
        _fps_kernel,
        out_shape=jax.ShapeDtypeStruct((B, NUM_GROUP, 3), jnp.float32),
        grid=(B,),
        in_specs=[pl.BlockSpec((1, 3, NROW, NCOL), lambda b: (b, 0, 0, 0))],
        out_specs=pl.BlockSpec((1, NUM_GROUP, 3), lambda b: (b, 0, 0)),
        compiler_params=pltpu.CompilerParams(
            dimension_semantics=("parallel",)),
    )(xr)


def kernel(xyz):
    B, N, _ = xyz.shape
    center = _fps_centers(xyz)
    d2 = (
        jnp.sum(center ** 2, axis=-1)[:, :, None]
        + jnp.sum(xyz ** 2, axis=-1)[:, None, :]
        - 2.0 * jnp.einsum("bgd,bnd->bgn", center, xyz)
    )
    _, idx = jax.lax.top_k(-d2, GROUP_SIZE)
    neighborhood_org = jax.vmap(lambda p, i: p[i])(xyz, idx)
    neighborhood = neighborhood_org - center[:, :, None, :]
    return (neighborhood, center, neighborhood_org)


# speedup vs baseline: 6.4487x; 4.9528x over previous
import jax
import jax.numpy as jnp
from jax import lax
from jax.experimental import pallas as pl
from jax.experimental.pallas import tpu as pltpu
from jax.experimental.pallas import tpu_sc as plsc

NUM_GROUP = 512
GROUP_SIZE = 32
NROW = 64
NCOL = 128
N = NROW * NCOL
GB = 128
NLEV = 5
WROW = 32
CAP = 192


def _fps_kernel(xr_ref, center_ref):
    xa = xr_ref[0, 0]
    ya = xr_ref[0, 1]
    za = xr_ref[0, 2]
    flat = (lax.broadcasted_iota(jnp.int32, (NROW, NCOL), 0) * NCOL
            + lax.broadcasted_iota(jnp.int32, (NROW, NCOL), 1))
    lane = lax.broadcasted_iota(jnp.int32, (1, NCOL), 1)

    def extract(coord, r, l):
        row = xr_ref[0, coord, pl.ds(r, 1), :]
        return jnp.sum(jnp.where(lane == l, row, 0.0))

    cx0 = extract(0, 0, 0)
    cy0 = extract(1, 0, 0)
    cz0 = extract(2, 0, 0)
    center_ref[0, pl.ds(0, 1), pl.ds(0, 1)] = cx0[None, None]
    center_ref[0, pl.ds(0, 1), pl.ds(1, 1)] = cy0[None, None]
    center_ref[0, pl.ds(0, 1), pl.ds(2, 1)] = cz0[None, None]

    def body(i, carry):
        dmin, cx, cy, cz = carry
        dx = xa - cx
        dy = ya - cy
        dz = za - cz
        d = (dx * dx + dy * dy) + dz * dz
        dmin = jnp.minimum(dmin, d)
        m = jnp.max(dmin)
        cand = jnp.where(dmin == m, flat, jnp.int32(1 << 30))
        nxt = jnp.min(cand)
        r = nxt // NCOL
        l = nxt - r * NCOL
        ncx = extract(0, r, l)
        ncy = extract(1, r, l)
        ncz = extract(2, r, l)
        center_ref[0, pl.ds(i, 1), pl.ds(0, 1)] = ncx[None, None]
        center_ref[0, pl.ds(i, 1), pl.ds(1, 1)] = ncy[None, None]
        center_ref[0, pl.ds(i, 1), pl.ds(2, 1)] = ncz[None, None]
        return (dmin, ncx, ncy, ncz)

    dmin0 = jnp.full((NROW, NCOL), jnp.inf, dtype=jnp.float32)
    lax.fori_loop(1, NUM_GROUP, body, (dmin0, cx0, cy0, cz0))


def _fps_centers(xr):
    B = xr.shape[0]
    return pl.pallas_call(
        _fps_kernel,
        out_shape=jax.ShapeDtypeStruct((B, NUM_GROUP, 3), jnp.float32),
        grid=(B,),
        in_specs=[pl.BlockSpec((1, 3, NROW, NCOL), lambda b: (b, 0, 0, 0))],
        out_specs=pl.BlockSpec((1, NUM_GROUP, 3), lambda b: (b, 0, 0)),
        compiler_params=pltpu.CompilerParams(
            dimension_semantics=("parallel",)),
    )(xr)


def _summary_kernel(xt_ref, c_ref, *refs):
    m_refs = refs[:NLEV]
    a_refs = refs[NLEV:2 * NLEV]
    t_ref = refs[2 * NLEV]
    d2_ref = refs[2 * NLEV + 1]

    c = c_ref[0]
    cn = (c[:, 0:1] * c[:, 0:1] + c[:, 1:2] * c[:, 1:2]
          + c[:, 2:3] * c[:, 2:3])
    xt = xt_ref[0]
    x = xt[0:1, :]
    y = xt[1:2, :]
    z = xt[2:3, :]
    xn = (x * x + y * y) + z * z
    mm = jax.lax.dot_general(c, xt, (((1,), (0,)), ((), ())),
                             preferred_element_type=jnp.float32)
    d2 = (cn + xn) - 2.0 * mm
    for d in range(NROW):
        d2_ref[d] = d2[:, d * NCOL:(d + 1) * NCOL]

    BIG = jnp.float32(jnp.inf)
    Ms = []
    As = []
    for k in range(NLEV):
        def pm(d, mk, _As=tuple(As)):
            v = d2_ref[d]
            if _As:
                excl = _As[0] == d
                for aa in _As[1:]:
                    excl = excl | (aa == d)
                v = jnp.where(excl, BIG, v)
            return jnp.minimum(mk, v)

        mk = lax.fori_loop(0, NROW, pm, jnp.full((GB, NCOL), BIG), unroll=2)

        def pa(d, ak, _As=tuple(As), _mk=mk):
            v = d2_ref[d]
            hit = v == _mk
            for aa in _As:
                hit = hit & (aa != d)
            return jnp.minimum(ak, jnp.where(hit, d, NROW))

        ak = lax.fori_loop(0, NROW, pa,
                           jnp.full((GB, NCOL), NROW, jnp.int32), unroll=2)
        m_refs[k][0] = mk
        a_refs[k][0] = ak
        Ms.append(mk)
        As.append(ak)

    s = jnp.concatenate([Ms[0], Ms[1]], axis=1)
    W = 2 * NCOL
    lanew = lax.broadcasted_iota(jnp.int32, (1, W), 1)
    k = 2
    while k <= W:
        j = k // 2
        while j >= 1:
            bit0 = (lanew & j) == 0
            if k < W:
                dirm = (lanew & k) == 0
            else:
                dirm = jnp.full((1, W), True)
            rm = pltpu.roll(s, W - j, axis=1)
            rp = pltpu.roll(s, j, axis=1)
            p = jnp.where(bit0, rm, rp)
            keepmin = jnp.logical_not(jnp.logical_xor(bit0, dirm))
            s = jnp.where(keepmin, jnp.minimum(s, p), jnp.maximum(s, p))
            j //= 2
        k *= 2
    t_ref[0] = s[:, 31:32]


def _summaries(xt3, center):
    B = xt3.shape[0]
    G = NUM_GROUP
    ngb = G // GB
    fblk = pl.BlockSpec((1, GB, NCOL), lambda b, g: (b, g, 0))
    return pl.pallas_call(
        _summary_kernel,
        out_shape=tuple(
            [jax.ShapeDtypeStruct((B, G, NCOL), jnp.float32)] * NLEV
            + [jax.ShapeDtypeStruct((B, G, NCOL), jnp.int32)] * NLEV
            + [jax.ShapeDtypeStruct((B, G, 1), jnp.float32)]
        ),
        grid=(B, ngb),
        in_specs=[
            pl.BlockSpec((1, 3, N), lambda b, g: (b, 0, 0)),
            pl.BlockSpec((1, GB, 3), lambda b, g: (b, g, 0)),
        ],
        out_specs=tuple([fblk] * (2 * NLEV)
                        + [pl.BlockSpec((1, GB, 1), lambda b, g: (b, g, 0))]),
        scratch_shapes=[pltpu.VMEM((NROW, GB, NCOL), jnp.float32)],
        compiler_params=pltpu.CompilerParams(
            dimension_semantics=("parallel", "arbitrary")),
    )(xt3, center)


def _sc_body(xt_ref, cf_ref, *refs):
    m_hbm = refs[:NLEV]
    a_hbm = refs[NLEV:2 * NLEV]
    nb_ref = refs[2 * NLEV]
    nbo_ref = refs[2 * NLEV + 1]
    scr = refs[2 * NLEV + 2:]
    x_s, y_s, z_s, ctr_s = scr[:4]
    m_s = scr[4:4 + NLEV]
    a_s = scr[4 + NLEV:4 + 2 * NLEV]
    bufv, bufi, idxk, outn, outo = scr[4 + 2 * NLEV:]

    nc = 2
    wid = lax.axis_index("s") * nc + lax.axis_index("c")
    b = wid
    iota = lax.broadcasted_iota(jnp.int32, (16,), 0)
    INF = jnp.float32(jnp.inf)

    pltpu.sync_copy(xt_ref.at[3 * b + 0], x_s)
    pltpu.sync_copy(xt_ref.at[3 * b + 1], y_s)
    pltpu.sync_copy(xt_ref.at[3 * b + 2], z_s)

    def window(w, _):
        row0 = b * NUM_GROUP + w * WROW
        for k in range(NLEV):
            pltpu.sync_copy(m_hbm[k].at[pl.ds(row0, WROW)], m_s[k])
            pltpu.sync_copy(a_hbm[k].at[pl.ds(row0, WROW)], a_s[k])
        pltpu.sync_copy(cf_ref.at[pl.ds(row0, WROW)], ctr_s)

        def row(r, _):
            cvec = ctr_s[r, pl.ds(0, 16)]
            cx = cvec[0]
            cy = cvec[1]
            cz = cvec[2]
            tg = cvec[3]

            for kk in range(CAP // 16):
                bufv[pl.ds(kk * 16, 16)] = jnp.full((16,), INF)

            off = jnp.zeros((16,), jnp.int32)
            for c in range(8):
                lanevec = c * 16 + iota
                for k in range(NLEV):
                    mkv = m_s[k][r, pl.ds(c * 16, 16)]
                    akv = a_s[k][r, pl.ds(c * 16, 16)]
                    take = mkv <= tg
                    nidx = akv * NCOL + lanevec
                    cs = plsc.cumsum(take.astype(jnp.int32))
                    pos = off + cs - 1
                    pok = take & (pos < CAP)
                    plsc.store_scatter(bufv, [pos], mkv, mask=pok)
                    plsc.store_scatter(bufi, [pos], nidx, mask=pok)
                    off = off + plsc.all_reduce_population_count(take)

            offs = jnp.max(off)

            def extract(nv):
                vs = [bufv[pl.ds(kk * 16, 16)] for kk in range(nv)]
                ivs = [bufi[pl.ds(kk * 16, 16)] for kk in range(nv)]

                def step(j, carry):
                    vs = list(carry[:nv])
                    ivs = list(carry[nv:])
                    m = vs[0]
                    for kk in range(1, nv):
                        m = jnp.minimum(m, vs[kk])
                    ms = jnp.min(m)
                    ci = jnp.where(vs[0] == ms, ivs[0], jnp.int32(1 << 30))
                    for kk in range(1, nv):
                        ci = jnp.minimum(
                            ci, jnp.where(vs[kk] == ms, ivs[kk],
                                          jnp.int32(1 << 30)))
                    isc = jnp.min(ci)
                    plsc.store_scatter(idxk, [jnp.full((16,), j, jnp.int32)],
                                       jnp.full((16,), isc, jnp.int32),
                                       mask=iota == 0)
                    for kk in range(nv):
                        sel = (vs[kk] == ms) & (ivs[kk] == isc)
                        vs[kk] = jnp.where(sel, INF, vs[kk])
                    return tuple(vs) + tuple(ivs)

                lax.fori_loop(0, GROUP_SIZE, step, tuple(vs) + tuple(ivs))

            def fast():
                extract(4)

            def slow():
                extract(CAP // 16)

            jax.lax.cond(offs <= 64, fast, slow)

            for h in range(2):
                iv = idxk[pl.ds(h * 16, 16)]
                xg = plsc.load_gather(x_s, [iv])
                yg = plsc.load_gather(y_s, [iv])
                zg = plsc.load_gather(z_s, [iv])
                rvec = jnp.full((16,), r, jnp.int32)
                pvec = h * 48 + (iota * 3)
                plsc.store_scatter(outo, [rvec, pvec], xg)
                plsc.store_scatter(outo, [rvec, pvec + 1], yg)
                plsc.store_scatter(outo, [rvec, pvec + 2], zg)
                plsc.store_scatter(outn, [rvec, pvec], xg - cx)
                plsc.store_scatter(outn, [rvec, pvec + 1], yg - cy)
                plsc.store_scatter(outn, [rvec, pvec + 2], zg - cz)
            return 0

        lax.fori_loop(0, WROW, row, 0)
        pltpu.sync_copy(outn, nb_ref.at[pl.ds(row0, WROW)])
        pltpu.sync_copy(outo, nbo_ref.at[pl.ds(row0, WROW)])
        return 0

    lax.fori_loop(0, NUM_GROUP // WROW, window, 0)


def _sc_select(xt2, cf, ms, as_):
    BG = cf.shape[0]
    mesh = plsc.VectorSubcoreMesh(core_axis_name="c", subcore_axis_name="s")
    kfn = pl.kernel(
        _sc_body,
        out_type=(
            jax.ShapeDtypeStruct((BG, 3 * GROUP_SIZE), jnp.float32),
            jax.ShapeDtypeStruct((BG, 3 * GROUP_SIZE), jnp.float32),
        ),
        mesh=mesh,
        compiler_params=pltpu.CompilerParams(needs_layout_passes=False),
        scratch_types=(
            [pltpu.VMEM((N,), jnp.float32)] * 3
            + [pltpu.VMEM((WROW, 16), jnp.float32)]
            + [pltpu.VMEM((WROW, NCOL), jnp.float32)] * NLEV
            + [pltpu.VMEM((WROW, NCOL), jnp.int32)] * NLEV
            + [
                pltpu.VMEM((CAP,), jnp.float32),
                pltpu.VMEM((CAP,), jnp.int32),
                pltpu.VMEM((GROUP_SIZE,), jnp.int32),
                pltpu.VMEM((WROW, 3 * GROUP_SIZE), jnp.float32),
                pltpu.VMEM((WROW, 3 * GROUP_SIZE), jnp.float32),
            ]
        ),
    )
    return kfn(xt2, cf, *ms, *as_)


def kernel(xyz):
    B = xyz.shape[0]
    G = NUM_GROUP
    M = GROUP_SIZE
    xr = xyz.transpose(0, 2, 1).reshape(B, 3, NROW, NCOL)
    center = _fps_centers(xr)
    xt3 = xr.reshape(B, 3, N)
    outs = _summaries(xt3, center)
    ms = [o.reshape(B * G, NCOL) for o in outs[:NLEV]]
    as_ = [o.reshape(B * G, NCOL) for o in outs[NLEV:2 * NLEV]]
    t = outs[2 * NLEV]
    xt2 = xr.reshape(B * 3, N)
    cf = jnp.concatenate(
        [center.reshape(B * G, 3), t.reshape(B * G, 1),
         jnp.zeros((B * G, 12), jnp.float32)], axis=1)
    nb, nbo = _sc_select(xt2, cf, ms, as_)
    neighborhood = nb.reshape(B, G, M, 3)
    neighborhood_org = nbo.reshape(B, G, M, 3)
    return (neighborhood, center, neighborhood_org)


# speedup vs baseline: 11.9443x; 1.8522x over previous
import jax
import jax.numpy as jnp
from jax import lax
from jax.experimental import pallas as pl
from jax.experimental.pallas import tpu as pltpu
from jax.experimental.pallas import tpu_sc as plsc

NUM_GROUP = 512
GROUP_SIZE = 32
NROW = 64
NCOL = 128
N = NROW * NCOL
GB = 128
NLEV = 5
WROW = 32
CAP = 192


NBATCH = 16


def _fps_kernel(xr_ref, center_ref):
    xa = xr_ref[0, :, 0]
    ya = xr_ref[0, :, 1]
    za = xr_ref[0, :, 2]
    flat = (lax.broadcasted_iota(jnp.int32, (NBATCH, NROW, NCOL), 1) * NCOL
            + lax.broadcasted_iota(jnp.int32, (NBATCH, NROW, NCOL), 2))
    lane = lax.broadcasted_iota(jnp.int32, (1, NCOL), 1)

    def extract(bb, coord, r, l):
        row = xr_ref[0, bb, coord, pl.ds(r, 1), :]
        return jnp.sum(jnp.where(lane == l, row, 0.0))

    cx0 = xa[:, 0:1, 0:1]
    cy0 = ya[:, 0:1, 0:1]
    cz0 = za[:, 0:1, 0:1]
    for bb in range(NBATCH):
        center_ref[0, bb, pl.ds(0, 1), pl.ds(0, 1)] = cx0[bb]
        center_ref[0, bb, pl.ds(0, 1), pl.ds(1, 1)] = cy0[bb]
        center_ref[0, bb, pl.ds(0, 1), pl.ds(2, 1)] = cz0[bb]

    def body(i, carry):
        dmin, cx, cy, cz = carry
        dx = xa - cx
        dy = ya - cy
        dz = za - cz
        d = (dx * dx + dy * dy) + dz * dz
        dmin = jnp.minimum(dmin, d)
        m = jnp.max(dmin, axis=(1, 2), keepdims=True)
        cand = jnp.where(dmin == m, flat, jnp.int32(1 << 30))
        nxt = jnp.min(cand, axis=(1, 2), keepdims=True)
        ncxs, ncys, nczs = [], [], []
        for bb in range(NBATCH):
            nb = nxt[bb, 0, 0]
            r = nb // NCOL
            l = nb - r * NCOL
            ncx = extract(bb, 0, r, l)
            ncy = extract(bb, 1, r, l)
            ncz = extract(bb, 2, r, l)
            center_ref[0, bb, pl.ds(i, 1), pl.ds(0, 1)] = ncx[None, None]
            center_ref[0, bb, pl.ds(i, 1), pl.ds(1, 1)] = ncy[None, None]
            center_ref[0, bb, pl.ds(i, 1), pl.ds(2, 1)] = ncz[None, None]
            ncxs.append(ncx[None, None, None])
            ncys.append(ncy[None, None, None])
            nczs.append(ncz[None, None, None])
        return (dmin, jnp.concatenate(ncxs), jnp.concatenate(ncys),
                jnp.concatenate(nczs))

    dmin0 = jnp.full((NBATCH, NROW, NCOL), jnp.inf, dtype=jnp.float32)
    lax.fori_loop(1, NUM_GROUP, body, (dmin0, cx0, cy0, cz0))


def _fps_centers(xr):
    B = xr.shape[0]
    nbl = B // NBATCH
    xr2 = xr.reshape(nbl, NBATCH, 3, NROW, NCOL)
    out = pl.pallas_call(
        _fps_kernel,
        out_shape=jax.ShapeDtypeStruct((nbl, NBATCH, NUM_GROUP, 3),
                                       jnp.float32),
        grid=(nbl,),
        in_specs=[pl.BlockSpec((1, NBATCH, 3, NROW, NCOL),
                               lambda b: (b, 0, 0, 0, 0))],
        out_specs=pl.BlockSpec((1, NBATCH, NUM_GROUP, 3),
                               lambda b: (b, 0, 0, 0)),
        compiler_params=pltpu.CompilerParams(
            dimension_semantics=("parallel",)),
    )(xr2)
    return out.reshape(B, NUM_GROUP, 3)


def _summary_kernel(xt_ref, c_ref, *refs):
    m_refs = refs[:NLEV]
    a_refs = refs[NLEV:2 * NLEV]
    t_ref = refs[2 * NLEV]
    d2_ref = refs[2 * NLEV + 1]

    c = c_ref[0]
    cn = (c[:, 0:1] * c[:, 0:1] + c[:, 1:2] * c[:, 1:2]
          + c[:, 2:3] * c[:, 2:3])
    xt = xt_ref[0]
    x = xt[0:1, :]
    y = xt[1:2, :]
    z = xt[2:3, :]
    xn = (x * x + y * y) + z * z
    mm = jax.lax.dot_general(c, xt, (((1,), (0,)), ((), ())),
                             preferred_element_type=jnp.float32)
    d2 = (cn + xn) - 2.0 * mm
    for d in range(NROW):
        d2_ref[d] = d2[:, d * NCOL:(d + 1) * NCOL]

    BIG = jnp.float32(jnp.inf)
    Ms = []
    As = []
    for k in range(NLEV):
        def pm(d, mk, _As=tuple(As)):
            v = d2_ref[d]
            if _As:
                excl = _As[0] == d
                for aa in _As[1:]:
                    excl = excl | (aa == d)
                v = jnp.where(excl, BIG, v)
            return jnp.minimum(mk, v)

        mk = lax.fori_loop(0, NROW, pm, jnp.full((GB, NCOL), BIG), unroll=2)

        def pa(d, ak, _As=tuple(As), _mk=mk):
            v = d2_ref[d]
            hit = v == _mk
            for aa in _As:
                hit = hit & (aa != d)
            return jnp.minimum(ak, jnp.where(hit, d, NROW))

        ak = lax.fori_loop(0, NROW, pa,
                           jnp.full((GB, NCOL), NROW, jnp.int32), unroll=2)
        m_refs[k][0] = mk
        a_refs[k][0] = ak
        Ms.append(mk)
        As.append(ak)

    s = jnp.concatenate([Ms[0], Ms[1]], axis=1)
    W = 2 * NCOL
    lanew = lax.broadcasted_iota(jnp.int32, (1, W), 1)
    k = 2
    while k <= W:
        j = k // 2
        while j >= 1:
            bit0 = (lanew & j) == 0
            if k < W:
                dirm = (lanew & k) == 0
            else:
                dirm = jnp.full((1, W), True)
            rm = pltpu.roll(s, W - j, axis=1)
            rp = pltpu.roll(s, j, axis=1)
            p = jnp.where(bit0, rm, rp)
            keepmin = jnp.logical_not(jnp.logical_xor(bit0, dirm))
            s = jnp.where(keepmin, jnp.minimum(s, p), jnp.maximum(s, p))
            j //= 2
        k *= 2
    t_ref[0] = s[:, 31:32]


def _summaries(xt3, center):
    B = xt3.shape[0]
    G = NUM_GROUP
    ngb = G // GB
    fblk = pl.BlockSpec((1, GB, NCOL), lambda b, g: (b, g, 0))
    return pl.pallas_call(
        _summary_kernel,
        out_shape=tuple(
            [jax.ShapeDtypeStruct((B, G, NCOL), jnp.float32)] * NLEV
            + [jax.ShapeDtypeStruct((B, G, NCOL), jnp.int32)] * NLEV
            + [jax.ShapeDtypeStruct((B, G, 1), jnp.float32)]
        ),
        grid=(B, ngb),
        in_specs=[
            pl.BlockSpec((1, 3, N), lambda b, g: (b, 0, 0)),
            pl.BlockSpec((1, GB, 3), lambda b, g: (b, g, 0)),
        ],
        out_specs=tuple([fblk] * (2 * NLEV)
                        + [pl.BlockSpec((1, GB, 1), lambda b, g: (b, g, 0))]),
        scratch_shapes=[pltpu.VMEM((NROW, GB, NCOL), jnp.float32)],
        compiler_params=pltpu.CompilerParams(
            dimension_semantics=("parallel", "arbitrary")),
    )(xt3, center)


def _sc_body(xt_ref, cf_ref, *refs):
    m_hbm = refs[:NLEV]
    a_hbm = refs[NLEV:2 * NLEV]
    nb_ref = refs[2 * NLEV]
    nbo_ref = refs[2 * NLEV + 1]
    scr = refs[2 * NLEV + 2:]
    x_s, y_s, z_s, ctr_s = scr[:4]
    m_s = scr[4:4 + NLEV]
    a_s = scr[4 + NLEV:4 + 2 * NLEV]
    bufv, bufi, idxk, outn, outo = scr[4 + 2 * NLEV:]

    nc = 2
    wid = lax.axis_index("s") * nc + lax.axis_index("c")
    b = wid
    iota = lax.broadcasted_iota(jnp.int32, (16,), 0)
    INF = jnp.float32(jnp.inf)

    pltpu.sync_copy(xt_ref.at[3 * b + 0], x_s)
    pltpu.sync_copy(xt_ref.at[3 * b + 1], y_s)
    pltpu.sync_copy(xt_ref.at[3 * b + 2], z_s)

    def window(w, _):
        row0 = b * NUM_GROUP + w * WROW
        for k in range(NLEV):
            pltpu.sync_copy(m_hbm[k].at[pl.ds(row0, WROW)], m_s[k])
            pltpu.sync_copy(a_hbm[k].at[pl.ds(row0, WROW)], a_s[k])
        pltpu.sync_copy(cf_ref.at[pl.ds(row0, WROW)], ctr_s)

        def row(r, _):
            cvec = ctr_s[r, pl.ds(0, 16)]
            cx = cvec[0]
            cy = cvec[1]
            cz = cvec[2]
            tg = cvec[3]

            for kk in range(CAP // 16):
                bufv[pl.ds(kk * 16, 16)] = jnp.full((16,), INF)

            off = jnp.zeros((16,), jnp.int32)
            for c in range(8):
                lanevec = c * 16 + iota
                for k in range(NLEV):
                    mkv = m_s[k][r, pl.ds(c * 16, 16)]
                    akv = a_s[k][r, pl.ds(c * 16, 16)]
                    take = mkv <= tg
                    nidx = akv * NCOL + lanevec
                    cs = plsc.cumsum(take.astype(jnp.int32))
                    pos = off + cs - 1
                    pok = take & (pos < CAP)
                    plsc.store_scatter(bufv, [pos], mkv, mask=pok)
                    plsc.store_scatter(bufi, [pos], nidx, mask=pok)
                    off = off + plsc.all_reduce_population_count(take)

            offs = jnp.max(off)

            def extract(nv):
                vs = [bufv[pl.ds(kk * 16, 16)] for kk in range(nv)]
                ivs = [bufi[pl.ds(kk * 16, 16)] for kk in range(nv)]

                def step(j, carry):
                    vs = list(carry[:nv])
                    ivs = list(carry[nv:])
                    m = vs[0]
                    for kk in range(1, nv):
                        m = jnp.minimum(m, vs[kk])
                    ms = jnp.min(m)
                    ci = jnp.where(vs[0] == ms, ivs[0], jnp.int32(1 << 30))
                    for kk in range(1, nv):
                        ci = jnp.minimum(
                            ci, jnp.where(vs[kk] == ms, ivs[kk],
                                          jnp.int32(1 << 30)))
                    isc = jnp.min(ci)
                    plsc.store_scatter(idxk, [jnp.full((16,), j, jnp.int32)],
                                       jnp.full((16,), isc, jnp.int32),
                                       mask=iota == 0)
                    for kk in range(nv):
                        sel = (vs[kk] == ms) & (ivs[kk] == isc)
                        vs[kk] = jnp.where(sel, INF, vs[kk])
                    return tuple(vs) + tuple(ivs)

                lax.fori_loop(0, GROUP_SIZE, step, tuple(vs) + tuple(ivs))

            def fast():
                extract(4)

            def slow():
                extract(CAP // 16)

            jax.lax.cond(offs <= 64, fast, slow)

            for h in range(2):
                iv = idxk[pl.ds(h * 16, 16)]
                xg = plsc.load_gather(x_s, [iv])
                yg = plsc.load_gather(y_s, [iv])
                zg = plsc.load_gather(z_s, [iv])
                rvec = jnp.full((16,), r, jnp.int32)
                pvec = h * 48 + (iota * 3)
                plsc.store_scatter(outo, [rvec, pvec], xg)
                plsc.store_scatter(outo, [rvec, pvec + 1], yg)
                plsc.store_scatter(outo, [rvec, pvec + 2], zg)
                plsc.store_scatter(outn, [rvec, pvec], xg - cx)
                plsc.store_scatter(outn, [rvec, pvec + 1], yg - cy)
                plsc.store_scatter(outn, [rvec, pvec + 2], zg - cz)
            return 0

        lax.fori_loop(0, WROW, row, 0)
        pltpu.sync_copy(outn, nb_ref.at[pl.ds(row0, WROW)])
        pltpu.sync_copy(outo, nbo_ref.at[pl.ds(row0, WROW)])
        return 0

    lax.fori_loop(0, NUM_GROUP // WROW, window, 0)


def _sc_select(xt2, cf, ms, as_):
    BG = cf.shape[0]
    mesh = plsc.VectorSubcoreMesh(core_axis_name="c", subcore_axis_name="s")
    kfn = pl.kernel(
        _sc_body,
        out_type=(
            jax.ShapeDtypeStruct((BG, 3 * GROUP_SIZE), jnp.float32),
            jax.ShapeDtypeStruct((BG, 3 * GROUP_SIZE), jnp.float32),
        ),
        mesh=mesh,
        compiler_params=pltpu.CompilerParams(needs_layout_passes=False),
        scratch_types=(
            [pltpu.VMEM((N,), jnp.float32)] * 3
            + [pltpu.VMEM((WROW, 16), jnp.float32)]
            + [pltpu.VMEM((WROW, NCOL), jnp.float32)] * NLEV
            + [pltpu.VMEM((WROW, NCOL), jnp.int32)] * NLEV
            + [
                pltpu.VMEM((CAP,), jnp.float32),
                pltpu.VMEM((CAP,), jnp.int32),
                pltpu.VMEM((GROUP_SIZE,), jnp.int32),
                pltpu.VMEM((WROW, 3 * GROUP_SIZE), jnp.float32),
                pltpu.VMEM((WROW, 3 * GROUP_SIZE), jnp.float32),
            ]
        ),
    )
    return kfn(xt2, cf, *ms, *as_)


def kernel(xyz):
    B = xyz.shape[0]
    G = NUM_GROUP
    M = GROUP_SIZE
    xr = xyz.transpose(0, 2, 1).reshape(B, 3, NROW, NCOL)
    center = _fps_centers(xr)
    xt3 = xr.reshape(B, 3, N)
    outs = _summaries(xt3, center)
    ms = [o.reshape(B * G, NCOL) for o in outs[:NLEV]]
    as_ = [o.reshape(B * G, NCOL) for o in outs[NLEV:2 * NLEV]]
    t = outs[2 * NLEV]
    xt2 = xr.reshape(B * 3, N)
    cf = jnp.concatenate(
        [center.reshape(B * G, 3), t.reshape(B * G, 1),
         jnp.zeros((B * G, 12), jnp.float32)], axis=1)
    nb, nbo = _sc_select(xt2, cf, ms, as_)
    neighborhood = nb.reshape(B, G, M, 3)
    neighborhood_org = nbo.reshape(B, G, M, 3)
    return (neighborhood, center, neighborhood_org)


# speedup vs baseline: 13.6417x; 1.1421x over previous
import jax
import jax.numpy as jnp
from jax import lax
from jax.experimental import pallas as pl
from jax.experimental.pallas import tpu as pltpu
from jax.experimental.pallas import tpu_sc as plsc

NUM_GROUP = 512
GROUP_SIZE = 32
NROW = 64
NCOL = 128
N = NROW * NCOL
GB = 128
NLEV = 5
WROW = 32
CAP = 192


NBATCH = 16


def _fps_kernel(xr_ref, center_ref):
    xa = xr_ref[0, :, 0]
    ya = xr_ref[0, :, 1]
    za = xr_ref[0, :, 2]
    flat = (lax.broadcasted_iota(jnp.int32, (NBATCH, NROW, NCOL), 1) * NCOL
            + lax.broadcasted_iota(jnp.int32, (NBATCH, NROW, NCOL), 2))
    lane = lax.broadcasted_iota(jnp.int32, (1, NCOL), 1)

    def extract(bb, coord, r, l):
        row = xr_ref[0, bb, coord, pl.ds(r, 1), :]
        return jnp.sum(jnp.where(lane == l, row, 0.0))

    cx0 = xa[:, 0:1, 0:1]
    cy0 = ya[:, 0:1, 0:1]
    cz0 = za[:, 0:1, 0:1]
    for bb in range(NBATCH):
        center_ref[0, bb, pl.ds(0, 1), pl.ds(0, 1)] = cx0[bb]
        center_ref[0, bb, pl.ds(0, 1), pl.ds(1, 1)] = cy0[bb]
        center_ref[0, bb, pl.ds(0, 1), pl.ds(2, 1)] = cz0[bb]

    def body(i, carry):
        dmin, cx, cy, cz = carry
        dx = xa - cx
        dy = ya - cy
        dz = za - cz
        d = (dx * dx + dy * dy) + dz * dz
        dmin = jnp.minimum(dmin, d)
        m = jnp.max(dmin, axis=(1, 2), keepdims=True)
        cand = jnp.where(dmin == m, flat, jnp.int32(1 << 30))
        nxt = jnp.min(cand, axis=(1, 2), keepdims=True)
        ncxs, ncys, nczs = [], [], []
        for bb in range(NBATCH):
            nb = nxt[bb, 0, 0]
            r = nb // NCOL
            l = nb - r * NCOL
            ncx = extract(bb, 0, r, l)
            ncy = extract(bb, 1, r, l)
            ncz = extract(bb, 2, r, l)
            center_ref[0, bb, pl.ds(i, 1), pl.ds(0, 1)] = ncx[None, None]
            center_ref[0, bb, pl.ds(i, 1), pl.ds(1, 1)] = ncy[None, None]
            center_ref[0, bb, pl.ds(i, 1), pl.ds(2, 1)] = ncz[None, None]
            ncxs.append(ncx[None, None, None])
            ncys.append(ncy[None, None, None])
            nczs.append(ncz[None, None, None])
        return (dmin, jnp.concatenate(ncxs), jnp.concatenate(ncys),
                jnp.concatenate(nczs))

    dmin0 = jnp.full((NBATCH, NROW, NCOL), jnp.inf, dtype=jnp.float32)
    lax.fori_loop(1, NUM_GROUP, body, (dmin0, cx0, cy0, cz0))


def _fps_centers(xr):
    B = xr.shape[0]
    nbl = B // NBATCH
    xr2 = xr.reshape(nbl, NBATCH, 3, NROW, NCOL)
    out = pl.pallas_call(
        _fps_kernel,
        out_shape=jax.ShapeDtypeStruct((nbl, NBATCH, NUM_GROUP, 3),
                                       jnp.float32),
        grid=(nbl,),
        in_specs=[pl.BlockSpec((1, NBATCH, 3, NROW, NCOL),
                               lambda b: (b, 0, 0, 0, 0))],
        out_specs=pl.BlockSpec((1, NBATCH, NUM_GROUP, 3),
                               lambda b: (b, 0, 0, 0)),
        compiler_params=pltpu.CompilerParams(
            dimension_semantics=("parallel",)),
    )(xr2)
    return out.reshape(B, NUM_GROUP, 3)


def _summary_kernel(xt_ref, c_ref, *refs):
    m_refs = refs[:NLEV]
    a_refs = refs[NLEV:2 * NLEV]
    t_ref = refs[2 * NLEV]
    d2_ref = refs[2 * NLEV + 1]

    c = c_ref[0]
    cn = (c[:, 0:1] * c[:, 0:1] + c[:, 1:2] * c[:, 1:2]
          + c[:, 2:3] * c[:, 2:3])
    xt = xt_ref[0]
    x = xt[0:1, :]
    y = xt[1:2, :]
    z = xt[2:3, :]
    xn = (x * x + y * y) + z * z
    mm = jax.lax.dot_general(c, xt, (((1,), (0,)), ((), ())),
                             preferred_element_type=jnp.float32)
    d2 = (cn + xn) - 2.0 * mm
    for d in range(NROW):
        d2_ref[d] = d2[:, d * NCOL:(d + 1) * NCOL]

    BIG = jnp.float32(jnp.inf)
    Ms = []
    As = []
    for k in range(NLEV):
        def pboth(d, carry, _As=tuple(As)):
            mk, ak = carry
            v = d2_ref[d]
            if _As:
                excl = _As[0] == d
                for aa in _As[1:]:
                    excl = excl | (aa == d)
                v = jnp.where(excl, BIG, v)
            better = v < mk
            return (jnp.where(better, v, mk), jnp.where(better, d, ak))

        mk, ak = lax.fori_loop(
            0, NROW, pboth,
            (jnp.full((GB, NCOL), BIG),
             jnp.full((GB, NCOL), NROW, jnp.int32)), unroll=2)
        m_refs[k][0] = mk
        a_refs[k][0] = ak
        Ms.append(mk)
        As.append(ak)

    s = jnp.concatenate([Ms[0], Ms[1]], axis=1)
    W = 2 * NCOL
    lanew = lax.broadcasted_iota(jnp.int32, (1, W), 1)
    k = 2
    while k <= W:
        j = k // 2
        while j >= 1:
            bit0 = (lanew & j) == 0
            if k < W:
                dirm = (lanew & k) == 0
            else:
                dirm = jnp.full((1, W), True)
            rm = pltpu.roll(s, W - j, axis=1)
            rp = pltpu.roll(s, j, axis=1)
            p = jnp.where(bit0, rm, rp)
            keepmin = jnp.logical_not(jnp.logical_xor(bit0, dirm))
            s = jnp.where(keepmin, jnp.minimum(s, p), jnp.maximum(s, p))
            j //= 2
        k *= 2
    t_ref[0] = s[:, 31:32]


def _summaries(xt3, center):
    B = xt3.shape[0]
    G = NUM_GROUP
    ngb = G // GB
    fblk = pl.BlockSpec((1, GB, NCOL), lambda b, g: (b, g, 0))
    return pl.pallas_call(
        _summary_kernel,
        out_shape=tuple(
            [jax.ShapeDtypeStruct((B, G, NCOL), jnp.float32)] * NLEV
            + [jax.ShapeDtypeStruct((B, G, NCOL), jnp.int32)] * NLEV
            + [jax.ShapeDtypeStruct((B, G, 1), jnp.float32)]
        ),
        grid=(B, ngb),
        in_specs=[
            pl.BlockSpec((1, 3, N), lambda b, g: (b, 0, 0)),
            pl.BlockSpec((1, GB, 3), lambda b, g: (b, g, 0)),
        ],
        out_specs=tuple([fblk] * (2 * NLEV)
                        + [pl.BlockSpec((1, GB, 1), lambda b, g: (b, g, 0))]),
        scratch_shapes=[pltpu.VMEM((NROW, GB, NCOL), jnp.float32)],
        compiler_params=pltpu.CompilerParams(
            dimension_semantics=("parallel", "arbitrary")),
    )(xt3, center)


def _sc_body(xt_ref, cf_ref, *refs):
    m_hbm = refs[:NLEV]
    a_hbm = refs[NLEV:2 * NLEV]
    nb_ref = refs[2 * NLEV]
    nbo_ref = refs[2 * NLEV + 1]
    scr = refs[2 * NLEV + 2:]
    x_s, y_s, z_s, ctr_s = scr[:4]
    m_s = scr[4:4 + NLEV]
    a_s = scr[4 + NLEV:4 + 2 * NLEV]
    bufv, bufi, idxk, outn, outo = scr[4 + 2 * NLEV:]

    nc = 2
    wid = lax.axis_index("s") * nc + lax.axis_index("c")
    b = wid
    iota = lax.broadcasted_iota(jnp.int32, (16,), 0)
    INF = jnp.float32(jnp.inf)

    pltpu.sync_copy(xt_ref.at[3 * b + 0], x_s)
    pltpu.sync_copy(xt_ref.at[3 * b + 1], y_s)
    pltpu.sync_copy(xt_ref.at[3 * b + 2], z_s)

    def window(w, _):
        row0 = b * NUM_GROUP + w * WROW
        for k in range(NLEV):
            pltpu.sync_copy(m_hbm[k].at[pl.ds(row0, WROW)], m_s[k])
            pltpu.sync_copy(a_hbm[k].at[pl.ds(row0, WROW)], a_s[k])
        pltpu.sync_copy(cf_ref.at[pl.ds(row0, WROW)], ctr_s)

        def row(r, _):
            cvec = ctr_s[r, pl.ds(0, 16)]
            cx = cvec[0]
            cy = cvec[1]
            cz = cvec[2]
            tg = cvec[3]

            for kk in range(CAP // 16):
                bufv[pl.ds(kk * 16, 16)] = jnp.full((16,), INF)

            off = jnp.zeros((16,), jnp.int32)
            for c in range(8):
                lanevec = c * 16 + iota
                for k in range(NLEV):
                    mkv = m_s[k][r, pl.ds(c * 16, 16)]
                    akv = a_s[k][r, pl.ds(c * 16, 16)]
                    take = mkv <= tg
                    nidx = akv * NCOL + lanevec
                    cs = plsc.cumsum(take.astype(jnp.int32))
                    pos = off + cs - 1
                    pok = take & (pos < CAP)
                    plsc.store_scatter(bufv, [pos], mkv, mask=pok)
                    plsc.store_scatter(bufi, [pos], nidx, mask=pok)
                    off = off + plsc.all_reduce_population_count(take)

            offs = jnp.max(off)

            def extract(nv):
                vs = [bufv[pl.ds(kk * 16, 16)] for kk in range(nv)]
                ivs = [bufi[pl.ds(kk * 16, 16)] for kk in range(nv)]

                def step(j, carry):
                    vs = list(carry[:nv])
                    ivs = list(carry[nv:])
                    m = vs[0]
                    for kk in range(1, nv):
                        m = jnp.minimum(m, vs[kk])
                    ms = jnp.min(m)
                    ci = jnp.where(vs[0] == ms, ivs[0], jnp.int32(1 << 30))
                    for kk in range(1, nv):
                        ci = jnp.minimum(
                            ci, jnp.where(vs[kk] == ms, ivs[kk],
                                          jnp.int32(1 << 30)))
                    isc = jnp.min(ci)
                    plsc.store_scatter(idxk, [jnp.full((16,), j, jnp.int32)],
                                       jnp.full((16,), isc, jnp.int32),
                                       mask=iota == 0)
                    for kk in range(nv):
                        sel = (vs[kk] == ms) & (ivs[kk] == isc)
                        vs[kk] = jnp.where(sel, INF, vs[kk])
                    return tuple(vs) + tuple(ivs)

                lax.fori_loop(0, GROUP_SIZE, step, tuple(vs) + tuple(ivs))

            def fast():
                extract(4)

            def slow():
                extract(CAP // 16)

            jax.lax.cond(offs <= 64, fast, slow)

            for h in range(2):
                iv = idxk[pl.ds(h * 16, 16)]
                xg = plsc.load_gather(x_s, [iv])
                yg = plsc.load_gather(y_s, [iv])
                zg = plsc.load_gather(z_s, [iv])
                rvec = jnp.full((16,), r, jnp.int32)
                pvec = h * 48 + (iota * 3)
                plsc.store_scatter(outo, [rvec, pvec], xg)
                plsc.store_scatter(outo, [rvec, pvec + 1], yg)
                plsc.store_scatter(outo, [rvec, pvec + 2], zg)
                plsc.store_scatter(outn, [rvec, pvec], xg - cx)
                plsc.store_scatter(outn, [rvec, pvec + 1], yg - cy)
                plsc.store_scatter(outn, [rvec, pvec + 2], zg - cz)
            return 0

        lax.fori_loop(0, WROW, row, 0)
        pltpu.sync_copy(outn, nb_ref.at[pl.ds(row0, WROW)])
        pltpu.sync_copy(outo, nbo_ref.at[pl.ds(row0, WROW)])
        return 0

    lax.fori_loop(0, NUM_GROUP // WROW, window, 0)


def _sc_select(xt2, cf, ms, as_):
    BG = cf.shape[0]
    mesh = plsc.VectorSubcoreMesh(core_axis_name="c", subcore_axis_name="s")
    kfn = pl.kernel(
        _sc_body,
        out_type=(
            jax.ShapeDtypeStruct((BG, 3 * GROUP_SIZE), jnp.float32),
            jax.ShapeDtypeStruct((BG, 3 * GROUP_SIZE), jnp.float32),
        ),
        mesh=mesh,
        compiler_params=pltpu.CompilerParams(needs_layout_passes=False),
        scratch_types=(
            [pltpu.VMEM((N,), jnp.float32)] * 3
            + [pltpu.VMEM((WROW, 16), jnp.float32)]
            + [pltpu.VMEM((WROW, NCOL), jnp.float32)] * NLEV
            + [pltpu.VMEM((WROW, NCOL), jnp.int32)] * NLEV
            + [
                pltpu.VMEM((CAP,), jnp.float32),
                pltpu.VMEM((CAP,), jnp.int32),
                pltpu.VMEM((GROUP_SIZE,), jnp.int32),
                pltpu.VMEM((WROW, 3 * GROUP_SIZE), jnp.float32),
                pltpu.VMEM((WROW, 3 * GROUP_SIZE), jnp.float32),
            ]
        ),
    )
    return kfn(xt2, cf, *ms, *as_)


def kernel(xyz):
    B = xyz.shape[0]
    G = NUM_GROUP
    M = GROUP_SIZE
    xr = xyz.transpose(0, 2, 1).reshape(B, 3, NROW, NCOL)
    center = _fps_centers(xr)
    xt3 = xr.reshape(B, 3, N)
    outs = _summaries(xt3, center)
    ms = [o.reshape(B * G, NCOL) for o in outs[:NLEV]]
    as_ = [o.reshape(B * G, NCOL) for o in outs[NLEV:2 * NLEV]]
    t = outs[2 * NLEV]
    xt2 = xr.reshape(B * 3, N)
    cf = jnp.concatenate(
        [center.reshape(B * G, 3), t.reshape(B * G, 1),
         jnp.zeros((B * G, 12), jnp.float32)], axis=1)
    nb, nbo = _sc_select(xt2, cf, ms, as_)
    neighborhood = nb.reshape(B, G, M, 3)
    neighborhood_org = nbo.reshape(B, G, M, 3)
    return (neighborhood, center, neighborhood_org)
